# baseline (speedup 1.0000x reference)
import functools

import jax
import jax.numpy as jnp
from jax import lax
from jax.experimental import pallas as pl
from jax.experimental.pallas import tpu as pltpu
from jax.experimental.pallas import tpu_sc as plsc

EMB = 32
SEQ = 200
BATCH = 4096
NW = 32
BT = BATCH // NW
LANES = 16
NBUF = 8


def _sc_embed(xv, emb_table, pos_table):
  mesh = plsc.VectorSubcoreMesh(core_axis_name="c", subcore_axis_name="s")

  @functools.partial(
      pl.kernel,
      out_type=jax.ShapeDtypeStruct((SEQ, EMB // 8, NW, 8, BT), jnp.float32),
      mesh=mesh,
      scratch_types=[
          pltpu.VMEM((SEQ, EMB), jnp.float32),
          pltpu.VMEM((SEQ // 8, 8, BT), jnp.int32),
          pltpu.VMEM((NBUF, BT, EMB), jnp.float32),
          pltpu.VMEM((NBUF, EMB, BT + 1), jnp.float32),
          [pltpu.SemaphoreType.DMA] * NBUF,
          [pltpu.SemaphoreType.DMA] * NBUF,
      ],
      compiler_params=pltpu.CompilerParams(
          use_tc_tiling_on_sc=False, needs_layout_passes=False),
  )
  def k(table_hbm, xv_hbm, pos_hbm, out_hbm, pos_v, idx_v, rows_v, tile_v,
        gsems, osems):
    wid = lax.axis_index("s") * 2 + lax.axis_index("c")

    pltpu.sync_copy(pos_hbm, pos_v)
    pltpu.sync_copy(xv_hbm.at[:, wid], idx_v)

    ea = lax.iota(jnp.int32, LANES)
    eb = ea + LANES

    def idx_at(s):
      return idx_v.at[lax.div(s, 8), lax.rem(s, 8)]

    def fire_gather(s, b):
      pltpu.async_copy(table_hbm.at[idx_at(s)], rows_v.at[b], gsems[b])

    for s in range(NBUF - 1):
      fire_gather(s, s)

    @pl.loop(0, SEQ, step=NBUF)
    def _ring(g):
      for b in range(NBUF):
        s = g + b

        pltpu.make_async_copy(
            table_hbm.at[idx_at(s)], rows_v.at[b], gsems[b]).wait()

        @pl.when(s + NBUF - 1 < SEQ)
        def _start_next():
          fire_gather(s + NBUF - 1, (b + NBUF - 1) % NBUF)

        @pl.when(s >= NBUF)
        def _drain_out():
          for et in range(EMB // 8):
            pltpu.make_async_copy(
                tile_v.at[b, pl.ds(et * 8, 8), pl.ds(0, BT)],
                out_hbm.at[s - NBUF, et, wid], osems[b]).wait()

        pa = pos_v[s, pl.ds(0, LANES)]
        pb = pos_v[s, pl.ds(LANES, LANES)]

        @plsc.parallel_loop(0, BT, unroll=32)
        def _tr(r):
          rv = jnp.full((LANES,), r, jnp.int32)
          va = rows_v[b, r, pl.ds(0, LANES)] + pa
          vb = rows_v[b, r, pl.ds(LANES, LANES)] + pb
          plsc.store_scatter(tile_v.at[b], [ea, rv], va)
          plsc.store_scatter(tile_v.at[b], [eb, rv], vb)

        for et in range(EMB // 8):
          pltpu.async_copy(
              tile_v.at[b, pl.ds(et * 8, 8), pl.ds(0, BT)],
              out_hbm.at[s, et, wid], osems[b])

    for b in range(NBUF):
      s = SEQ - NBUF + b
      for et in range(EMB // 8):
        pltpu.make_async_copy(
            tile_v.at[b, pl.ds(et * 8, 8), pl.ds(0, BT)],
            out_hbm.at[s, et, wid], osems[b]).wait()

  return k(emb_table, xv, pos_table)


@jax.jit
def kernel(x, emb_table, pos_table):
  xv = x.T.reshape(SEQ // 8, 8, NW, BT).transpose(0, 2, 1, 3)
  out5 = _sc_embed(xv, emb_table, pos_table)
  return out5.transpose(2, 4, 0, 1, 3).reshape(BATCH, SEQ, EMB)


# speedup vs baseline: 1.4008x; 1.4008x over previous
import functools

import jax
import jax.numpy as jnp
from jax import lax
from jax.experimental import pallas as pl
from jax.experimental.pallas import tpu as pltpu
from jax.experimental.pallas import tpu_sc as plsc

EMB = 32
SEQ = 200
BATCH = 4096
NW = 32
BT = BATCH // NW
LANES = 16
NBUF = 8


def _sc_embed(xv, emb_table, pos_table):
  mesh = plsc.VectorSubcoreMesh(core_axis_name="c", subcore_axis_name="s")

  @functools.partial(
      pl.kernel,
      out_type=jax.ShapeDtypeStruct((SEQ, EMB // 8, NW, 8, BT), jnp.float32),
      mesh=mesh,
      scratch_types=[
          pltpu.VMEM((SEQ, EMB), jnp.float32),
          pltpu.VMEM((SEQ // 8, 8, BT), jnp.int32),
          pltpu.VMEM((NBUF, BT, EMB), jnp.float32),
          pltpu.VMEM((NBUF, EMB, BT + 1), jnp.float32),
          [pltpu.SemaphoreType.DMA] * NBUF,
          [pltpu.SemaphoreType.DMA] * NBUF,
      ],
      compiler_params=pltpu.CompilerParams(
          use_tc_tiling_on_sc=False, needs_layout_passes=False),
  )
  def k(table_hbm, xv_hbm, pos_hbm, out_hbm, pos_v, idx_v, rows_v, tile_v,
        gsems, osems):
    wid = lax.axis_index("s") * 2 + lax.axis_index("c")

    pltpu.sync_copy(pos_hbm, pos_v)
    pltpu.sync_copy(xv_hbm.at[:, wid], idx_v)

    ea = lax.iota(jnp.int32, LANES)
    eb = ea + LANES

    def idx_at(s):
      return idx_v.at[lax.div(s, 8), lax.rem(s, 8)]

    def fire_gather(s, b):
      pltpu.async_copy(table_hbm.at[idx_at(s)], rows_v.at[b], gsems[b])

    for s in range(NBUF - 1):
      fire_gather(s, s)

    @pl.loop(0, SEQ, step=NBUF)
    def _ring(g):
      for b in range(NBUF):
        s = g + b

        pltpu.make_async_copy(
            table_hbm.at[idx_at(s)], rows_v.at[b], gsems[b]).wait()

        @pl.when(s + NBUF - 1 < SEQ)
        def _start_next():
          fire_gather(s + NBUF - 1, (b + NBUF - 1) % NBUF)

        @pl.when(s >= NBUF)
        def _drain_out():
          for et in range(EMB // 8):
            pltpu.make_async_copy(
                tile_v.at[b, pl.ds(et * 8, 8), pl.ds(0, BT)],
                out_hbm.at[s - NBUF, et, wid], osems[b]).wait()

        pa = pos_v[s, pl.ds(0, LANES)]
        pb = pos_v[s, pl.ds(LANES, LANES)]

        @plsc.parallel_loop(0, BT, unroll=16)
        def _tr(r):
          rv = jnp.full((LANES,), r, jnp.int32)
          va = rows_v[b, r, pl.ds(0, LANES)] + pa
          vb = rows_v[b, r, pl.ds(LANES, LANES)] + pb
          plsc.store_scatter(tile_v.at[b], [ea, rv], va)
          plsc.store_scatter(tile_v.at[b], [eb, rv], vb)

        for et in range(EMB // 8):
          pltpu.async_copy(
              tile_v.at[b, pl.ds(et * 8, 8), pl.ds(0, BT)],
              out_hbm.at[s, et, wid], osems[b])

    for b in range(NBUF):
      s = SEQ - NBUF + b
      for et in range(EMB // 8):
        pltpu.make_async_copy(
            tile_v.at[b, pl.ds(et * 8, 8), pl.ds(0, BT)],
            out_hbm.at[s, et, wid], osems[b]).wait()

  return k(emb_table, xv, pos_table)


@jax.jit
def kernel(x, emb_table, pos_table):
  xv = x.T.reshape(SEQ // 8, 8, NW, BT).transpose(0, 2, 1, 3)
  out5 = _sc_embed(xv, emb_table, pos_table)
  return out5.transpose(2, 4, 0, 1, 3).reshape(BATCH, SEQ, EMB)
